# baseline (speedup 1.0000x reference)
import functools

import jax
import jax.numpy as jnp
from jax import lax
from jax.experimental import pallas as pl
from jax.experimental.pallas import tpu as pltpu
from jax.experimental.pallas import tpu_sc as plsc

NC = 2
NS = 16
CH = 128
BN = 1024
NG = 64


def _sc_degree(n_pad, e_pad):
    cpt = e_pad // (NC * NS * CH)
    rpt = n_pad // NS
    rchunks = rpt // CH
    mesh = plsc.VectorSubcoreMesh(
        core_axis_name="c", subcore_axis_name="s", num_cores=NC)

    @functools.partial(
        pl.kernel, mesh=mesh,
        out_type=jax.ShapeDtypeStruct((NC, n_pad, 128), jnp.float32),
        scratch_types=[
            pltpu.VMEM((CH,), jnp.int32),
            pltpu.VMEM((CH,), jnp.int32),
            pltpu.VMEM((CH, 128), jnp.float32),
            pltpu.VMEM((CH, 128), jnp.float32),
            pltpu.VMEM_SHARED((n_pad, 128), jnp.float32),
            pltpu.SemaphoreType.DMA,
            pltpu.SemaphoreType.DMA,
        ],
    )
    def deg_kernel(dst_hbm, out_hbm, didx0, didx1, ones_v, rows, acc,
                   ssem0, ssem1):
        c = lax.axis_index("c")
        s = lax.axis_index("s")
        z16 = jnp.zeros((16,), jnp.float32)
        o16 = jnp.ones((16,), jnp.float32)

        def fill(i, carry):
            for j in range(8):
                rows[i, pl.ds(j * 16, 16)] = z16
                ones_v[i, pl.ds(j * 16, 16)] = o16
            return carry
        lax.fori_loop(0, CH, fill, 0)
        iota16 = lax.iota(jnp.int32, 16)
        for j in range(8):
            didx0[pl.ds(j * 16, 16)] = s * rpt + j * 16 + iota16
            didx1[pl.ds(j * 16, 16)] = s * rpt + CH + j * 16 + iota16

        def zero_chunk(k, carry):
            pltpu.sync_copy(rows, acc.at[pl.ds(s * rpt + k * CH, CH)])
            return carry
        lax.fori_loop(0, rchunks, zero_chunk, 0)
        plsc.subcore_barrier()

        pltpu.async_copy(rows, acc.at[didx0], ssem0, add=True)
        pltpu.async_copy(rows, acc.at[didx1], ssem1, add=True)

        tile_base = (c * NS + s) * (cpt * CH)

        def pair_step(g, carry):
            base = tile_base + g * (2 * CH)
            pltpu.make_async_copy(ones_v, acc.at[didx0], ssem0).wait()
            pltpu.sync_copy(dst_hbm.at[pl.ds(base, CH)], didx0)
            pltpu.async_copy(ones_v, acc.at[didx0], ssem0, add=True)
            pltpu.make_async_copy(ones_v, acc.at[didx1], ssem1).wait()
            pltpu.sync_copy(dst_hbm.at[pl.ds(base + CH, CH)], didx1)
            pltpu.async_copy(ones_v, acc.at[didx1], ssem1, add=True)
            return carry
        lax.fori_loop(0, cpt // 2, pair_step, 0)
        pltpu.make_async_copy(ones_v, acc.at[didx0], ssem0).wait()
        pltpu.make_async_copy(ones_v, acc.at[didx1], ssem1).wait()

        plsc.subcore_barrier()

        def readout(k, carry):
            pltpu.sync_copy(acc.at[pl.ds(s * rpt + k * CH, CH)], rows)
            pltpu.sync_copy(rows, out_hbm.at[c, pl.ds(s * rpt + k * CH, CH)])
            return carry
        lax.fori_loop(0, rchunks, readout, 0)

    return deg_kernel


def _sc_aggregate(n_pad, e_pad):
    cpt = e_pad // (NC * NS * CH)
    rpt = n_pad // NS
    rchunks = rpt // CH
    mesh = plsc.VectorSubcoreMesh(
        core_axis_name="c", subcore_axis_name="s", num_cores=NC)

    @functools.partial(
        pl.kernel, mesh=mesh,
        out_type=jax.ShapeDtypeStruct((NC, n_pad, 128), jnp.float32),
        scratch_types=[
            pltpu.VMEM((CH,), jnp.int32),
            pltpu.VMEM((CH,), jnp.int32),
            pltpu.VMEM((CH, 128), jnp.float32),
            pltpu.VMEM_SHARED((n_pad, 128), jnp.float32),
            pltpu.SemaphoreType.DMA,
        ],
    )
    def agg_kernel(y_hbm, src_hbm, dst_hbm, out_hbm, sidx, didx, rows, acc,
                   sem):
        c = lax.axis_index("c")
        s = lax.axis_index("s")
        z16 = jnp.zeros((16,), jnp.float32)

        def fill_zero(i, carry):
            for j in range(8):
                rows[i, pl.ds(j * 16, 16)] = z16
            return carry
        lax.fori_loop(0, CH, fill_zero, 0)

        def zero_chunk(k, carry):
            pltpu.sync_copy(rows, acc.at[pl.ds(s * rpt + k * CH, CH)])
            return carry
        lax.fori_loop(0, rchunks, zero_chunk, 0)
        plsc.subcore_barrier()

        tile_base = (c * NS + s) * (cpt * CH)

        def edge_step(i, carry):
            b = tile_base + i * CH
            pltpu.sync_copy(src_hbm.at[pl.ds(b, CH)], sidx)
            pltpu.sync_copy(dst_hbm.at[pl.ds(b, CH)], didx)
            pltpu.async_copy(y_hbm.at[sidx], rows, sem).wait()
            pltpu.sync_copy(rows, acc.at[didx], add=True)
            return carry
        lax.fori_loop(0, cpt, edge_step, 0)

        plsc.subcore_barrier()

        def readout(k, carry):
            pltpu.sync_copy(acc.at[pl.ds(s * rpt + k * CH, CH)], rows)
            pltpu.sync_copy(rows, out_hbm.at[c, pl.ds(s * rpt + k * CH, CH)])
            return carry
        lax.fori_loop(0, rchunks, readout, 0)

    return agg_kernel


def _dinv_of(degp_blk):
    deg = jnp.sum(degp_blk[:, :, 0:1], axis=0) + 1.0
    return lax.rsqrt(deg)


def _tc_first(n_pad):
    grid = n_pad // BN

    def body(x_ref, w_ref, degp_ref, y_ref, dinv_ref):
        dinv = _dinv_of(degp_ref[...])
        y_ref[...] = jnp.dot(x_ref[...], w_ref[...],
                             preferred_element_type=jnp.float32) * dinv
        dinv_ref[...] = jnp.broadcast_to(dinv, (BN, 8))

    return pl.pallas_call(
        body,
        grid=(grid,),
        in_specs=[
            pl.BlockSpec((BN, 128), lambda i: (i, 0)),
            pl.BlockSpec((128, 128), lambda i: (0, 0)),
            pl.BlockSpec((NC, BN, 128), lambda i: (0, i, 0)),
        ],
        out_specs=[
            pl.BlockSpec((BN, 128), lambda i: (i, 0)),
            pl.BlockSpec((BN, 8), lambda i: (i, 0)),
        ],
        out_shape=[
            jax.ShapeDtypeStruct((n_pad, 128), jnp.float32),
            jax.ShapeDtypeStruct((n_pad, 8), jnp.float32),
        ],
    )


def _tc_mid(n_pad):
    grid = n_pad // BN

    def body(p_ref, y1_ref, dinv8_ref, b1_ref, w2_ref, y2_ref):
        dinv = dinv8_ref[:, 0:1]
        srow = (p_ref[0] + p_ref[1] + y1_ref[...]) * dinv
        h1 = jnp.maximum(srow + b1_ref[...], 0.0)
        y2_ref[...] = jnp.dot(h1, w2_ref[...],
                              preferred_element_type=jnp.float32) * dinv

    return pl.pallas_call(
        body,
        grid=(grid,),
        in_specs=[
            pl.BlockSpec((NC, BN, 128), lambda i: (0, i, 0)),
            pl.BlockSpec((BN, 128), lambda i: (i, 0)),
            pl.BlockSpec((BN, 8), lambda i: (i, 0)),
            pl.BlockSpec((1, 128), lambda i: (0, 0)),
            pl.BlockSpec((128, 128), lambda i: (0, 0)),
        ],
        out_specs=pl.BlockSpec((BN, 128), lambda i: (i, 0)),
        out_shape=jax.ShapeDtypeStruct((n_pad, 128), jnp.float32),
    )


def _tc_final(n_pad, n_out):
    grid = n_pad // BN

    def body(p_ref, y2_ref, dinv8_ref, b2_ref, batch_ref, fw1_ref, fb1_ref,
             fw2_ref, fb2_ref, out_ref, sums, cnt):
        i = pl.program_id(0)

        @pl.when(i == 0)
        def _():
            sums[...] = jnp.zeros_like(sums)
            cnt[...] = jnp.zeros_like(cnt)

        dinv = dinv8_ref[:, 0:1]
        srow = (p_ref[0] + p_ref[1] + y2_ref[...]) * dinv
        h2 = jnp.maximum(srow + b2_ref[...], 0.0)

        row_ids = lax.broadcasted_iota(jnp.int32, (NG, BN), 0)
        bmat = jnp.broadcast_to(batch_ref[...], (NG, BN))
        mask = (row_ids == bmat).astype(jnp.float32)
        sums[...] += jnp.dot(mask, h2, preferred_element_type=jnp.float32)
        cnt[...] += jnp.sum(mask, axis=1, keepdims=True)

        @pl.when(i == grid - 1)
        def _():
            g = sums[...] / jnp.maximum(cnt[...], 1.0)
            z1 = jnp.maximum(
                jnp.dot(g, fw1_ref[...],
                        preferred_element_type=jnp.float32) + fb1_ref[...],
                0.0)
            z2 = jnp.dot(z1, fw2_ref[...],
                         preferred_element_type=jnp.float32) + fb2_ref[...]
            m = jnp.max(z2, axis=1, keepdims=True)
            e = z2 - m
            out_ref[...] = e - jnp.log(
                jnp.sum(jnp.exp(e), axis=1, keepdims=True))

    return pl.pallas_call(
        body,
        grid=(grid,),
        in_specs=[
            pl.BlockSpec((NC, BN, 128), lambda i: (0, i, 0)),
            pl.BlockSpec((BN, 128), lambda i: (i, 0)),
            pl.BlockSpec((BN, 8), lambda i: (i, 0)),
            pl.BlockSpec((1, 128), lambda i: (0, 0)),
            pl.BlockSpec((1, BN), lambda i: (0, i)),
            pl.BlockSpec((128, 128), lambda i: (0, 0)),
            pl.BlockSpec((1, 128), lambda i: (0, 0)),
            pl.BlockSpec((128, n_out), lambda i: (0, 0)),
            pl.BlockSpec((1, n_out), lambda i: (0, 0)),
        ],
        out_specs=pl.BlockSpec((NG, n_out), lambda i: (0, 0)),
        out_shape=jax.ShapeDtypeStruct((NG, n_out), jnp.float32),
        scratch_shapes=[
            pltpu.VMEM((NG, 128), jnp.float32),
            pltpu.VMEM((NG, 1), jnp.float32),
        ],
    )


def kernel(x, edge_index, batch, W1, b1, W2, b2, fW1, fb1, fW2, fb2):
    n, _ = x.shape
    e = edge_index.shape[1]
    n_out = fW2.shape[1]

    grp = NS * CH
    n_pad = -(-(n + 1) // (NS * CH)) * (NS * CH)
    n_pad = -(-n_pad // BN) * BN
    e_grp = NC * NS * CH * 2
    e_pad = -(-e // e_grp) * e_grp
    del grp

    dummy = jnp.int32(n)
    src_p = jnp.concatenate(
        [edge_index[0], jnp.full((e_pad - e,), dummy, jnp.int32)])
    dst_p = jnp.concatenate(
        [edge_index[1], jnp.full((e_pad - e,), dummy, jnp.int32)])
    x_pad = jnp.pad(x, ((0, n_pad - n), (0, 0)))
    batch2 = jnp.concatenate(
        [batch, jnp.full((n_pad - n,), NG, jnp.int32)]).reshape(1, n_pad)

    degp = _sc_degree(n_pad, e_pad)(dst_p)
    y1, dinv8 = _tc_first(n_pad)(x_pad, W1, degp)
    agg = _sc_aggregate(n_pad, e_pad)
    p1 = agg(y1, src_p, dst_p)
    y2 = _tc_mid(n_pad)(p1, y1, dinv8, b1.reshape(1, -1), W2)
    p2 = agg(y2, src_p, dst_p)
    out = _tc_final(n_pad, n_out)(
        p2, y2, dinv8, b2.reshape(1, -1), batch2, fW1, fb1.reshape(1, -1),
        fW2, fb2.reshape(1, -1))
    return out


# speedup vs baseline: 2.1748x; 2.1748x over previous
import functools

import jax
import jax.numpy as jnp
from jax import lax
from jax.experimental import pallas as pl
from jax.experimental.pallas import tpu as pltpu
from jax.experimental.pallas import tpu_sc as plsc

NC = 2
NS = 16
CH = 128
BN = 1024
NG = 64


def _sc_degree(n_pad, e_pad):
    cpt = e_pad // (NC * NS * CH)
    rpt = n_pad // NS
    rchunks = rpt // CH
    mesh = plsc.VectorSubcoreMesh(
        core_axis_name="c", subcore_axis_name="s", num_cores=NC)

    @functools.partial(
        pl.kernel, mesh=mesh,
        out_type=jax.ShapeDtypeStruct((NC, n_pad, 128), jnp.float32),
        scratch_types=[
            pltpu.VMEM((CH,), jnp.int32),
            pltpu.VMEM((CH,), jnp.int32),
            pltpu.VMEM((CH, 128), jnp.float32),
            pltpu.VMEM((CH, 128), jnp.float32),
            pltpu.VMEM_SHARED((n_pad, 128), jnp.float32),
            pltpu.SemaphoreType.DMA,
            pltpu.SemaphoreType.DMA,
        ],
    )
    def deg_kernel(dst_hbm, out_hbm, didx0, didx1, ones_v, rows, acc,
                   ssem0, ssem1):
        c = lax.axis_index("c")
        s = lax.axis_index("s")
        z16 = jnp.zeros((16,), jnp.float32)
        o16 = jnp.ones((16,), jnp.float32)

        def fill(i, carry):
            for j in range(8):
                rows[i, pl.ds(j * 16, 16)] = z16
                ones_v[i, pl.ds(j * 16, 16)] = o16
            return carry
        lax.fori_loop(0, CH, fill, 0)
        iota16 = lax.iota(jnp.int32, 16)
        for j in range(8):
            didx0[pl.ds(j * 16, 16)] = s * rpt + j * 16 + iota16
            didx1[pl.ds(j * 16, 16)] = s * rpt + CH + j * 16 + iota16

        def zero_chunk(k, carry):
            pltpu.sync_copy(rows, acc.at[pl.ds(s * rpt + k * CH, CH)])
            return carry
        lax.fori_loop(0, rchunks, zero_chunk, 0)
        plsc.subcore_barrier()

        pltpu.async_copy(rows, acc.at[didx0], ssem0, add=True)
        pltpu.async_copy(rows, acc.at[didx1], ssem1, add=True)

        tile_base = (c * NS + s) * (cpt * CH)

        def pair_step(g, carry):
            base = tile_base + g * (2 * CH)
            pltpu.make_async_copy(ones_v, acc.at[didx0], ssem0).wait()
            pltpu.sync_copy(dst_hbm.at[pl.ds(base, CH)], didx0)
            pltpu.async_copy(ones_v, acc.at[didx0], ssem0, add=True)
            pltpu.make_async_copy(ones_v, acc.at[didx1], ssem1).wait()
            pltpu.sync_copy(dst_hbm.at[pl.ds(base + CH, CH)], didx1)
            pltpu.async_copy(ones_v, acc.at[didx1], ssem1, add=True)
            return carry
        lax.fori_loop(0, cpt // 2, pair_step, 0)
        pltpu.make_async_copy(ones_v, acc.at[didx0], ssem0).wait()
        pltpu.make_async_copy(ones_v, acc.at[didx1], ssem1).wait()

        plsc.subcore_barrier()

        def readout(k, carry):
            pltpu.sync_copy(acc.at[pl.ds(s * rpt + k * CH, CH)], rows)
            pltpu.sync_copy(rows, out_hbm.at[c, pl.ds(s * rpt + k * CH, CH)])
            return carry
        lax.fori_loop(0, rchunks, readout, 0)

    return deg_kernel


def _sc_aggregate(n_pad, e_pad):
    cpt = e_pad // (NC * NS * CH)
    rpt = n_pad // NS
    rchunks = rpt // CH
    mesh = plsc.VectorSubcoreMesh(
        core_axis_name="c", subcore_axis_name="s", num_cores=NC)

    @functools.partial(
        pl.kernel, mesh=mesh,
        out_type=jax.ShapeDtypeStruct((NC, n_pad, 128), jnp.float32),
        scratch_types=[
            pltpu.VMEM((CH,), jnp.int32),
            pltpu.VMEM((CH,), jnp.int32),
            pltpu.VMEM((CH, 128), jnp.float32),
            pltpu.VMEM_SHARED((n_pad, 128), jnp.float32),
            pltpu.SemaphoreType.DMA,
        ],
    )
    def agg_kernel(y_hbm, src_hbm, dst_hbm, out_hbm, sidx, didx, rows, acc,
                   sem):
        c = lax.axis_index("c")
        s = lax.axis_index("s")
        z16 = jnp.zeros((16,), jnp.float32)

        def fill_zero(i, carry):
            for j in range(8):
                rows[i, pl.ds(j * 16, 16)] = z16
            return carry
        lax.fori_loop(0, CH, fill_zero, 0)

        def zero_chunk(k, carry):
            pltpu.sync_copy(rows, acc.at[pl.ds(s * rpt + k * CH, CH)])
            return carry
        lax.fori_loop(0, rchunks, zero_chunk, 0)
        plsc.subcore_barrier()

        tile_base = (c * NS + s) * (cpt * CH)

        def edge_step(i, carry):
            b = tile_base + i * CH
            pltpu.sync_copy(src_hbm.at[pl.ds(b, CH)], sidx)
            pltpu.sync_copy(dst_hbm.at[pl.ds(b, CH)], didx)
            pltpu.async_copy(y_hbm.at[sidx], rows, sem).wait()
            pltpu.sync_copy(rows, acc.at[didx], add=True)
            return carry
        lax.fori_loop(0, cpt, edge_step, 0)

        plsc.subcore_barrier()

        def readout(k, carry):
            pltpu.sync_copy(acc.at[pl.ds(s * rpt + k * CH, CH)], rows)
            pltpu.sync_copy(rows, out_hbm.at[c, pl.ds(s * rpt + k * CH, CH)])
            return carry
        lax.fori_loop(0, rchunks, readout, 0)

    return agg_kernel


def _dinv_of(degp_blk):
    deg = jnp.sum(degp_blk[:, :, 0:1], axis=0) + 1.0
    return lax.rsqrt(deg)


def _tc_first(n_pad):
    grid = n_pad // BN

    def body(x_ref, w_ref, degp_ref, y_ref, dinv_ref):
        dinv = _dinv_of(degp_ref[...])
        y_ref[...] = jnp.dot(x_ref[...], w_ref[...],
                             preferred_element_type=jnp.float32) * dinv
        dinv_ref[...] = jnp.broadcast_to(dinv, (BN, 8))

    return pl.pallas_call(
        body,
        grid=(grid,),
        in_specs=[
            pl.BlockSpec((BN, 128), lambda i: (i, 0)),
            pl.BlockSpec((128, 128), lambda i: (0, 0)),
            pl.BlockSpec((NC, BN, 128), lambda i: (0, i, 0)),
        ],
        out_specs=[
            pl.BlockSpec((BN, 128), lambda i: (i, 0)),
            pl.BlockSpec((BN, 8), lambda i: (i, 0)),
        ],
        out_shape=[
            jax.ShapeDtypeStruct((n_pad, 128), jnp.float32),
            jax.ShapeDtypeStruct((n_pad, 8), jnp.float32),
        ],
    )


def _tc_mid(n_pad):
    grid = n_pad // BN

    def body(p_ref, y1_ref, dinv8_ref, b1_ref, w2_ref, y2_ref):
        dinv = dinv8_ref[:, 0:1]
        srow = (p_ref[0] + p_ref[1] + y1_ref[...]) * dinv
        h1 = jnp.maximum(srow + b1_ref[...], 0.0)
        y2_ref[...] = jnp.dot(h1, w2_ref[...],
                              preferred_element_type=jnp.float32) * dinv

    return pl.pallas_call(
        body,
        grid=(grid,),
        in_specs=[
            pl.BlockSpec((NC, BN, 128), lambda i: (0, i, 0)),
            pl.BlockSpec((BN, 128), lambda i: (i, 0)),
            pl.BlockSpec((BN, 8), lambda i: (i, 0)),
            pl.BlockSpec((1, 128), lambda i: (0, 0)),
            pl.BlockSpec((128, 128), lambda i: (0, 0)),
        ],
        out_specs=pl.BlockSpec((BN, 128), lambda i: (i, 0)),
        out_shape=jax.ShapeDtypeStruct((n_pad, 128), jnp.float32),
    )


def _tc_final(n_pad, n_out):
    grid = n_pad // BN

    def body(p_ref, y2_ref, dinv8_ref, b2_ref, batch_ref, fw1_ref, fb1_ref,
             fw2_ref, fb2_ref, out_ref, sums, cnt):
        i = pl.program_id(0)

        @pl.when(i == 0)
        def _():
            sums[...] = jnp.zeros_like(sums)
            cnt[...] = jnp.zeros_like(cnt)

        dinv = dinv8_ref[:, 0:1]
        srow = (p_ref[0] + p_ref[1] + y2_ref[...]) * dinv
        h2 = jnp.maximum(srow + b2_ref[...], 0.0)

        row_ids = lax.broadcasted_iota(jnp.int32, (NG, BN), 0)
        bmat = jnp.broadcast_to(batch_ref[...], (NG, BN))
        mask = (row_ids == bmat).astype(jnp.float32)
        sums[...] += jnp.dot(mask, h2, preferred_element_type=jnp.float32)
        cnt[...] += jnp.sum(mask, axis=1, keepdims=True)

        @pl.when(i == grid - 1)
        def _():
            g = sums[...] / jnp.maximum(cnt[...], 1.0)
            z1 = jnp.maximum(
                jnp.dot(g, fw1_ref[...],
                        preferred_element_type=jnp.float32) + fb1_ref[...],
                0.0)
            z2 = jnp.dot(z1, fw2_ref[...],
                         preferred_element_type=jnp.float32) + fb2_ref[...]
            m = jnp.max(z2, axis=1, keepdims=True)
            e = z2 - m
            out_ref[...] = e - jnp.log(
                jnp.sum(jnp.exp(e), axis=1, keepdims=True))

    return pl.pallas_call(
        body,
        grid=(grid,),
        in_specs=[
            pl.BlockSpec((NC, BN, 128), lambda i: (0, i, 0)),
            pl.BlockSpec((BN, 128), lambda i: (i, 0)),
            pl.BlockSpec((BN, 8), lambda i: (i, 0)),
            pl.BlockSpec((1, 128), lambda i: (0, 0)),
            pl.BlockSpec((1, BN), lambda i: (0, i)),
            pl.BlockSpec((128, 128), lambda i: (0, 0)),
            pl.BlockSpec((1, 128), lambda i: (0, 0)),
            pl.BlockSpec((128, n_out), lambda i: (0, 0)),
            pl.BlockSpec((1, n_out), lambda i: (0, 0)),
        ],
        out_specs=pl.BlockSpec((NG, n_out), lambda i: (0, 0)),
        out_shape=jax.ShapeDtypeStruct((NG, n_out), jnp.float32),
        scratch_shapes=[
            pltpu.VMEM((NG, 128), jnp.float32),
            pltpu.VMEM((NG, 1), jnp.float32),
        ],
    )


def kernel(x, edge_index, batch, W1, b1, W2, b2, fW1, fb1, fW2, fb2):
    n, _ = x.shape
    e = edge_index.shape[1]
    n_out = fW2.shape[1]

    grp = NS * CH
    n_pad = -(-(n + 1) // (NS * CH)) * (NS * CH)
    n_pad = -(-n_pad // BN) * BN
    e_grp = NC * NS * CH * 2
    e_pad = -(-e // e_grp) * e_grp
    del grp

    pad_idx = n + (jnp.arange(e_pad - e, dtype=jnp.int32) % (n_pad - n))
    src_p = jnp.concatenate([edge_index[0], pad_idx])
    dst_p = jnp.concatenate([edge_index[1], pad_idx])
    x_pad = jnp.pad(x, ((0, n_pad - n), (0, 0)))
    batch2 = jnp.concatenate(
        [batch, jnp.full((n_pad - n,), NG, jnp.int32)]).reshape(1, n_pad)

    degp = _sc_degree(n_pad, e_pad)(dst_p)
    y1, dinv8 = _tc_first(n_pad)(x_pad, W1, degp)
    agg = _sc_aggregate(n_pad, e_pad)
    p1 = agg(y1, src_p, dst_p)
    y2 = _tc_mid(n_pad)(p1, y1, dinv8, b1.reshape(1, -1), W2)
    p2 = agg(y2, src_p, dst_p)
    out = _tc_final(n_pad, n_out)(
        p2, y2, dinv8, b2.reshape(1, -1), batch2, fW1, fb1.reshape(1, -1),
        fW2, fb2.reshape(1, -1))
    return out


# speedup vs baseline: 3.0882x; 1.4200x over previous
import functools

import jax
import jax.numpy as jnp
from jax import lax
from jax.experimental import pallas as pl
from jax.experimental.pallas import tpu as pltpu
from jax.experimental.pallas import tpu_sc as plsc

NC = 2
NS = 16
CH = 128
BN = 1024
NG = 64


def _sc_degree(n_pad, e_pad):
    cpt = e_pad // (NC * NS * CH)
    rpt = n_pad // NS
    rchunks = rpt // CH
    mesh = plsc.VectorSubcoreMesh(
        core_axis_name="c", subcore_axis_name="s", num_cores=NC)

    @functools.partial(
        pl.kernel, mesh=mesh,
        out_type=jax.ShapeDtypeStruct((NC, n_pad, 128), jnp.float32),
        scratch_types=[
            pltpu.VMEM((CH,), jnp.int32),
            pltpu.VMEM((CH,), jnp.int32),
            pltpu.VMEM((CH, 128), jnp.float32),
            pltpu.VMEM((CH, 128), jnp.float32),
            pltpu.VMEM_SHARED((n_pad, 128), jnp.float32),
            pltpu.SemaphoreType.DMA,
            pltpu.SemaphoreType.DMA,
        ],
    )
    def deg_kernel(dst_hbm, out_hbm, didx0, didx1, ones_v, rows, acc,
                   ssem0, ssem1):
        c = lax.axis_index("c")
        s = lax.axis_index("s")
        z16 = jnp.zeros((16,), jnp.float32)
        o16 = jnp.ones((16,), jnp.float32)

        def fill(i, carry):
            for j in range(8):
                rows[i, pl.ds(j * 16, 16)] = z16
                ones_v[i, pl.ds(j * 16, 16)] = o16
            return carry
        lax.fori_loop(0, CH, fill, 0)
        iota16 = lax.iota(jnp.int32, 16)
        for j in range(8):
            didx0[pl.ds(j * 16, 16)] = s * rpt + j * 16 + iota16
            didx1[pl.ds(j * 16, 16)] = s * rpt + CH + j * 16 + iota16

        def zero_chunk(k, carry):
            pltpu.sync_copy(rows, acc.at[pl.ds(s * rpt + k * CH, CH)])
            return carry
        lax.fori_loop(0, rchunks, zero_chunk, 0)
        plsc.subcore_barrier()

        pltpu.async_copy(rows, acc.at[didx0], ssem0, add=True)
        pltpu.async_copy(rows, acc.at[didx1], ssem1, add=True)

        tile_base = (c * NS + s) * (cpt * CH)

        def pair_step(g, carry):
            base = tile_base + g * (2 * CH)
            pltpu.make_async_copy(ones_v, acc.at[didx0], ssem0).wait()
            pltpu.sync_copy(dst_hbm.at[pl.ds(base, CH)], didx0)
            pltpu.async_copy(ones_v, acc.at[didx0], ssem0, add=True)
            pltpu.make_async_copy(ones_v, acc.at[didx1], ssem1).wait()
            pltpu.sync_copy(dst_hbm.at[pl.ds(base + CH, CH)], didx1)
            pltpu.async_copy(ones_v, acc.at[didx1], ssem1, add=True)
            return carry
        lax.fori_loop(0, cpt // 2, pair_step, 0)
        pltpu.make_async_copy(ones_v, acc.at[didx0], ssem0).wait()
        pltpu.make_async_copy(ones_v, acc.at[didx1], ssem1).wait()

        plsc.subcore_barrier()

        def readout(k, carry):
            pltpu.sync_copy(acc.at[pl.ds(s * rpt + k * CH, CH)], rows)
            pltpu.sync_copy(rows, out_hbm.at[c, pl.ds(s * rpt + k * CH, CH)])
            return carry
        lax.fori_loop(0, rchunks, readout, 0)

    return deg_kernel


def _sc_aggregate(n_pad, e_pad):
    cpt = e_pad // (NC * NS * CH)
    rpt = n_pad // NS
    rchunks = rpt // CH
    mesh = plsc.VectorSubcoreMesh(
        core_axis_name="c", subcore_axis_name="s", num_cores=NC)

    @functools.partial(
        pl.kernel, mesh=mesh,
        out_type=jax.ShapeDtypeStruct((NC, n_pad, 128), jnp.float32),
        scratch_types=[
            pltpu.VMEM((CH,), jnp.int32),
            pltpu.VMEM((CH,), jnp.int32),
            pltpu.VMEM((CH,), jnp.int32),
            pltpu.VMEM((CH,), jnp.int32),
            pltpu.VMEM((CH, 128), jnp.float32),
            pltpu.VMEM((CH, 128), jnp.float32),
            pltpu.VMEM_SHARED((n_pad, 128), jnp.float32),
            pltpu.SemaphoreType.DMA,
            pltpu.SemaphoreType.DMA,
            pltpu.SemaphoreType.DMA,
            pltpu.SemaphoreType.DMA,
        ],
    )
    def agg_kernel(y_hbm, src_hbm, dst_hbm, out_hbm, sidx0, didx0, sidx1,
                   didx1, rows0, rows1, acc, gsem0, gsem1, ssem0, ssem1):
        c = lax.axis_index("c")
        s = lax.axis_index("s")
        z16 = jnp.zeros((16,), jnp.float32)

        def fill_zero(i, carry):
            for j in range(8):
                rows0[i, pl.ds(j * 16, 16)] = z16
                rows1[i, pl.ds(j * 16, 16)] = z16
            return carry
        lax.fori_loop(0, CH, fill_zero, 0)
        iota16 = lax.iota(jnp.int32, 16)
        for j in range(8):
            didx0[pl.ds(j * 16, 16)] = s * rpt + j * 16 + iota16
            didx1[pl.ds(j * 16, 16)] = s * rpt + CH + j * 16 + iota16

        def zero_chunk(k, carry):
            pltpu.sync_copy(rows0, acc.at[pl.ds(s * rpt + k * CH, CH)])
            return carry
        lax.fori_loop(0, rchunks, zero_chunk, 0)
        plsc.subcore_barrier()

        pltpu.async_copy(rows0, acc.at[didx0], ssem0, add=True)
        pltpu.async_copy(rows1, acc.at[didx1], ssem1, add=True)

        tile_base = (c * NS + s) * (cpt * CH)

        def pair_step(g, carry):
            base = tile_base + g * (2 * CH)
            pltpu.make_async_copy(rows0, acc.at[didx0], ssem0).wait()
            pltpu.sync_copy(src_hbm.at[pl.ds(base, CH)], sidx0)
            pltpu.sync_copy(dst_hbm.at[pl.ds(base, CH)], didx0)
            g0 = pltpu.async_copy(y_hbm.at[sidx0], rows0, gsem0)
            pltpu.make_async_copy(rows1, acc.at[didx1], ssem1).wait()
            pltpu.sync_copy(src_hbm.at[pl.ds(base + CH, CH)], sidx1)
            pltpu.sync_copy(dst_hbm.at[pl.ds(base + CH, CH)], didx1)
            g1 = pltpu.async_copy(y_hbm.at[sidx1], rows1, gsem1)
            g0.wait()
            pltpu.async_copy(rows0, acc.at[didx0], ssem0, add=True)
            g1.wait()
            pltpu.async_copy(rows1, acc.at[didx1], ssem1, add=True)
            return carry
        lax.fori_loop(0, cpt // 2, pair_step, 0)
        pltpu.make_async_copy(rows0, acc.at[didx0], ssem0).wait()
        pltpu.make_async_copy(rows1, acc.at[didx1], ssem1).wait()

        plsc.subcore_barrier()

        def readout(k, carry):
            pltpu.sync_copy(acc.at[pl.ds(s * rpt + k * CH, CH)], rows0)
            pltpu.sync_copy(rows0, out_hbm.at[c, pl.ds(s * rpt + k * CH, CH)])
            return carry
        lax.fori_loop(0, rchunks, readout, 0)

    return agg_kernel


def _dinv_of(degp_blk):
    deg = jnp.sum(degp_blk[:, :, 0:1], axis=0) + 1.0
    return lax.rsqrt(deg)


def _tc_first(n_pad):
    grid = n_pad // BN

    def body(x_ref, w_ref, degp_ref, y_ref, dinv_ref):
        dinv = _dinv_of(degp_ref[...])
        y_ref[...] = jnp.dot(x_ref[...], w_ref[...],
                             preferred_element_type=jnp.float32) * dinv
        dinv_ref[...] = jnp.broadcast_to(dinv, (BN, 8))

    return pl.pallas_call(
        body,
        grid=(grid,),
        in_specs=[
            pl.BlockSpec((BN, 128), lambda i: (i, 0)),
            pl.BlockSpec((128, 128), lambda i: (0, 0)),
            pl.BlockSpec((NC, BN, 128), lambda i: (0, i, 0)),
        ],
        out_specs=[
            pl.BlockSpec((BN, 128), lambda i: (i, 0)),
            pl.BlockSpec((BN, 8), lambda i: (i, 0)),
        ],
        out_shape=[
            jax.ShapeDtypeStruct((n_pad, 128), jnp.float32),
            jax.ShapeDtypeStruct((n_pad, 8), jnp.float32),
        ],
    )


def _tc_mid(n_pad):
    grid = n_pad // BN

    def body(p_ref, y1_ref, dinv8_ref, b1_ref, w2_ref, y2_ref):
        dinv = dinv8_ref[:, 0:1]
        srow = (p_ref[0] + p_ref[1] + y1_ref[...]) * dinv
        h1 = jnp.maximum(srow + b1_ref[...], 0.0)
        y2_ref[...] = jnp.dot(h1, w2_ref[...],
                              preferred_element_type=jnp.float32) * dinv

    return pl.pallas_call(
        body,
        grid=(grid,),
        in_specs=[
            pl.BlockSpec((NC, BN, 128), lambda i: (0, i, 0)),
            pl.BlockSpec((BN, 128), lambda i: (i, 0)),
            pl.BlockSpec((BN, 8), lambda i: (i, 0)),
            pl.BlockSpec((1, 128), lambda i: (0, 0)),
            pl.BlockSpec((128, 128), lambda i: (0, 0)),
        ],
        out_specs=pl.BlockSpec((BN, 128), lambda i: (i, 0)),
        out_shape=jax.ShapeDtypeStruct((n_pad, 128), jnp.float32),
    )


def _tc_final(n_pad, n_out):
    grid = n_pad // BN

    def body(p_ref, y2_ref, dinv8_ref, b2_ref, batch_ref, fw1_ref, fb1_ref,
             fw2_ref, fb2_ref, out_ref, sums, cnt):
        i = pl.program_id(0)

        @pl.when(i == 0)
        def _():
            sums[...] = jnp.zeros_like(sums)
            cnt[...] = jnp.zeros_like(cnt)

        dinv = dinv8_ref[:, 0:1]
        srow = (p_ref[0] + p_ref[1] + y2_ref[...]) * dinv
        h2 = jnp.maximum(srow + b2_ref[...], 0.0)

        row_ids = lax.broadcasted_iota(jnp.int32, (NG, BN), 0)
        bmat = jnp.broadcast_to(batch_ref[...], (NG, BN))
        mask = (row_ids == bmat).astype(jnp.float32)
        sums[...] += jnp.dot(mask, h2, preferred_element_type=jnp.float32)
        cnt[...] += jnp.sum(mask, axis=1, keepdims=True)

        @pl.when(i == grid - 1)
        def _():
            g = sums[...] / jnp.maximum(cnt[...], 1.0)
            z1 = jnp.maximum(
                jnp.dot(g, fw1_ref[...],
                        preferred_element_type=jnp.float32) + fb1_ref[...],
                0.0)
            z2 = jnp.dot(z1, fw2_ref[...],
                         preferred_element_type=jnp.float32) + fb2_ref[...]
            m = jnp.max(z2, axis=1, keepdims=True)
            e = z2 - m
            out_ref[...] = e - jnp.log(
                jnp.sum(jnp.exp(e), axis=1, keepdims=True))

    return pl.pallas_call(
        body,
        grid=(grid,),
        in_specs=[
            pl.BlockSpec((NC, BN, 128), lambda i: (0, i, 0)),
            pl.BlockSpec((BN, 128), lambda i: (i, 0)),
            pl.BlockSpec((BN, 8), lambda i: (i, 0)),
            pl.BlockSpec((1, 128), lambda i: (0, 0)),
            pl.BlockSpec((1, BN), lambda i: (0, i)),
            pl.BlockSpec((128, 128), lambda i: (0, 0)),
            pl.BlockSpec((1, 128), lambda i: (0, 0)),
            pl.BlockSpec((128, n_out), lambda i: (0, 0)),
            pl.BlockSpec((1, n_out), lambda i: (0, 0)),
        ],
        out_specs=pl.BlockSpec((NG, n_out), lambda i: (0, 0)),
        out_shape=jax.ShapeDtypeStruct((NG, n_out), jnp.float32),
        scratch_shapes=[
            pltpu.VMEM((NG, 128), jnp.float32),
            pltpu.VMEM((NG, 1), jnp.float32),
        ],
    )


def kernel(x, edge_index, batch, W1, b1, W2, b2, fW1, fb1, fW2, fb2):
    n, _ = x.shape
    e = edge_index.shape[1]
    n_out = fW2.shape[1]

    grp = NS * CH
    n_pad = -(-(n + 1) // (NS * CH)) * (NS * CH)
    n_pad = -(-n_pad // BN) * BN
    e_grp = NC * NS * CH * 2
    e_pad = -(-e // e_grp) * e_grp
    del grp

    pad_idx = n + (jnp.arange(e_pad - e, dtype=jnp.int32) % (n_pad - n))
    src_p = jnp.concatenate([edge_index[0], pad_idx])
    dst_p = jnp.concatenate([edge_index[1], pad_idx])
    x_pad = jnp.pad(x, ((0, n_pad - n), (0, 0)))
    batch2 = jnp.concatenate(
        [batch, jnp.full((n_pad - n,), NG, jnp.int32)]).reshape(1, n_pad)

    degp = _sc_degree(n_pad, e_pad)(dst_p)
    y1, dinv8 = _tc_first(n_pad)(x_pad, W1, degp)
    agg = _sc_aggregate(n_pad, e_pad)
    p1 = agg(y1, src_p, dst_p)
    y2 = _tc_mid(n_pad)(p1, y1, dinv8, b1.reshape(1, -1), W2)
    p2 = agg(y2, src_p, dst_p)
    out = _tc_final(n_pad, n_out)(
        p2, y2, dinv8, b2.reshape(1, -1), batch2, fW1, fb1.reshape(1, -1),
        fW2, fb2.reshape(1, -1))
    return out


# speedup vs baseline: 3.2447x; 1.0507x over previous
import functools

import jax
import jax.numpy as jnp
from jax import lax
from jax.experimental import pallas as pl
from jax.experimental.pallas import tpu as pltpu
from jax.experimental.pallas import tpu_sc as plsc

NC = 2
NS = 16
CH = 128
BN = 1024
NG = 64


def _sc_degree(n_pad, e_pad):
    cpt = e_pad // (NC * NS * CH)
    rpt = n_pad // NS
    rchunks = rpt // CH
    mesh = plsc.VectorSubcoreMesh(
        core_axis_name="c", subcore_axis_name="s", num_cores=NC)

    @functools.partial(
        pl.kernel, mesh=mesh,
        out_type=jax.ShapeDtypeStruct((NC, n_pad, 16), jnp.float32),
        scratch_types=[
            pltpu.VMEM((CH,), jnp.int32),
            pltpu.VMEM((CH,), jnp.int32),
            pltpu.VMEM((CH, 16), jnp.float32),
            pltpu.VMEM((CH, 16), jnp.float32),
            pltpu.VMEM_SHARED((n_pad, 16), jnp.float32),
            pltpu.SemaphoreType.DMA,
            pltpu.SemaphoreType.DMA,
        ],
    )
    def deg_kernel(dst_hbm, out_hbm, didx0, didx1, ones_v, rows, acc,
                   ssem0, ssem1):
        c = lax.axis_index("c")
        s = lax.axis_index("s")
        z16 = jnp.zeros((16,), jnp.float32)
        o16 = jnp.ones((16,), jnp.float32)

        def fill(i, carry):
            rows[i] = z16
            ones_v[i] = o16
            return carry
        lax.fori_loop(0, CH, fill, 0)
        iota16 = lax.iota(jnp.int32, 16)
        for j in range(8):
            didx0[pl.ds(j * 16, 16)] = s * rpt + j * 16 + iota16
            didx1[pl.ds(j * 16, 16)] = s * rpt + CH + j * 16 + iota16

        def zero_chunk(k, carry):
            pltpu.sync_copy(rows, acc.at[pl.ds(s * rpt + k * CH, CH)])
            return carry
        lax.fori_loop(0, rchunks, zero_chunk, 0)
        plsc.subcore_barrier()

        pltpu.async_copy(rows, acc.at[didx0], ssem0, add=True)
        pltpu.async_copy(rows, acc.at[didx1], ssem1, add=True)

        tile_base = (c * NS + s) * (cpt * CH)

        def pair_step(g, carry):
            base = tile_base + g * (2 * CH)
            pltpu.make_async_copy(ones_v, acc.at[didx0], ssem0).wait()
            pltpu.sync_copy(dst_hbm.at[pl.ds(base, CH)], didx0)
            pltpu.async_copy(ones_v, acc.at[didx0], ssem0, add=True)
            pltpu.make_async_copy(ones_v, acc.at[didx1], ssem1).wait()
            pltpu.sync_copy(dst_hbm.at[pl.ds(base + CH, CH)], didx1)
            pltpu.async_copy(ones_v, acc.at[didx1], ssem1, add=True)
            return carry
        lax.fori_loop(0, cpt // 2, pair_step, 0)
        pltpu.make_async_copy(ones_v, acc.at[didx0], ssem0).wait()
        pltpu.make_async_copy(ones_v, acc.at[didx1], ssem1).wait()

        plsc.subcore_barrier()

        def readout(k, carry):
            pltpu.sync_copy(acc.at[pl.ds(s * rpt + k * CH, CH)], rows)
            pltpu.sync_copy(rows, out_hbm.at[c, pl.ds(s * rpt + k * CH, CH)])
            return carry
        lax.fori_loop(0, rchunks, readout, 0)

    return deg_kernel


def _sc_aggregate(n_pad, e_pad):
    cpt = e_pad // (NC * NS * CH)
    rpt = n_pad // NS
    rchunks = rpt // CH
    mesh = plsc.VectorSubcoreMesh(
        core_axis_name="c", subcore_axis_name="s", num_cores=NC)

    @functools.partial(
        pl.kernel, mesh=mesh,
        out_type=jax.ShapeDtypeStruct((NC, n_pad, 128), jnp.float32),
        scratch_types=[
            pltpu.VMEM((CH,), jnp.int32),
            pltpu.VMEM((CH,), jnp.int32),
            pltpu.VMEM((CH,), jnp.int32),
            pltpu.VMEM((CH,), jnp.int32),
            pltpu.VMEM((CH, 128), jnp.float32),
            pltpu.VMEM((CH, 128), jnp.float32),
            pltpu.VMEM_SHARED((n_pad, 128), jnp.float32),
            pltpu.SemaphoreType.DMA,
            pltpu.SemaphoreType.DMA,
            pltpu.SemaphoreType.DMA,
            pltpu.SemaphoreType.DMA,
        ],
    )
    def agg_kernel(y_hbm, src_hbm, dst_hbm, out_hbm, sidx0, didx0, sidx1,
                   didx1, rows0, rows1, acc, gsem0, gsem1, ssem0, ssem1):
        c = lax.axis_index("c")
        s = lax.axis_index("s")
        z16 = jnp.zeros((16,), jnp.float32)

        def fill_zero(i, carry):
            for j in range(8):
                rows0[i, pl.ds(j * 16, 16)] = z16
                rows1[i, pl.ds(j * 16, 16)] = z16
            return carry
        lax.fori_loop(0, CH, fill_zero, 0)
        iota16 = lax.iota(jnp.int32, 16)
        for j in range(8):
            didx0[pl.ds(j * 16, 16)] = s * rpt + j * 16 + iota16
            didx1[pl.ds(j * 16, 16)] = s * rpt + CH + j * 16 + iota16

        def zero_chunk(k, carry):
            pltpu.sync_copy(rows0, acc.at[pl.ds(s * rpt + k * CH, CH)])
            return carry
        lax.fori_loop(0, rchunks, zero_chunk, 0)
        plsc.subcore_barrier()

        pltpu.async_copy(rows0, acc.at[didx0], ssem0, add=True)
        pltpu.async_copy(rows1, acc.at[didx1], ssem1, add=True)

        tile_base = (c * NS + s) * (cpt * CH)

        def pair_step(g, carry):
            base = tile_base + g * (2 * CH)
            pltpu.make_async_copy(rows0, acc.at[didx0], ssem0).wait()
            pltpu.sync_copy(src_hbm.at[pl.ds(base, CH)], sidx0)
            pltpu.sync_copy(dst_hbm.at[pl.ds(base, CH)], didx0)
            g0 = pltpu.async_copy(y_hbm.at[sidx0], rows0, gsem0)
            pltpu.make_async_copy(rows1, acc.at[didx1], ssem1).wait()
            pltpu.sync_copy(src_hbm.at[pl.ds(base + CH, CH)], sidx1)
            pltpu.sync_copy(dst_hbm.at[pl.ds(base + CH, CH)], didx1)
            g1 = pltpu.async_copy(y_hbm.at[sidx1], rows1, gsem1)
            g0.wait()
            pltpu.async_copy(rows0, acc.at[didx0], ssem0, add=True)
            g1.wait()
            pltpu.async_copy(rows1, acc.at[didx1], ssem1, add=True)
            return carry
        lax.fori_loop(0, cpt // 2, pair_step, 0)
        pltpu.make_async_copy(rows0, acc.at[didx0], ssem0).wait()
        pltpu.make_async_copy(rows1, acc.at[didx1], ssem1).wait()

        plsc.subcore_barrier()

        def readout(k, carry):
            pltpu.sync_copy(acc.at[pl.ds(s * rpt + k * CH, CH)], rows0)
            pltpu.sync_copy(rows0, out_hbm.at[c, pl.ds(s * rpt + k * CH, CH)])
            return carry
        lax.fori_loop(0, rchunks, readout, 0)

    return agg_kernel


def _dinv_of(degp_blk):
    deg = jnp.sum(degp_blk[:, :, 0:1], axis=0) + 1.0
    return lax.rsqrt(deg)


def _tc_first(n_pad):
    grid = n_pad // BN

    def body(x_ref, w_ref, degp_ref, y_ref, dinv_ref):
        dinv = _dinv_of(degp_ref[...])
        y_ref[...] = jnp.dot(x_ref[...], w_ref[...],
                             preferred_element_type=jnp.float32) * dinv
        dinv_ref[...] = jnp.broadcast_to(dinv, (BN, 8))

    return pl.pallas_call(
        body,
        grid=(grid,),
        in_specs=[
            pl.BlockSpec((BN, 128), lambda i: (i, 0)),
            pl.BlockSpec((128, 128), lambda i: (0, 0)),
            pl.BlockSpec((NC, BN, 16), lambda i: (0, i, 0)),
        ],
        out_specs=[
            pl.BlockSpec((BN, 128), lambda i: (i, 0)),
            pl.BlockSpec((BN, 8), lambda i: (i, 0)),
        ],
        out_shape=[
            jax.ShapeDtypeStruct((n_pad, 128), jnp.float32),
            jax.ShapeDtypeStruct((n_pad, 8), jnp.float32),
        ],
    )


def _tc_mid(n_pad):
    grid = n_pad // BN

    def body(p_ref, y1_ref, dinv8_ref, b1_ref, w2_ref, y2_ref):
        dinv = dinv8_ref[:, 0:1]
        srow = (p_ref[0] + p_ref[1] + y1_ref[...]) * dinv
        h1 = jnp.maximum(srow + b1_ref[...], 0.0)
        y2_ref[...] = jnp.dot(h1, w2_ref[...],
                              preferred_element_type=jnp.float32) * dinv

    return pl.pallas_call(
        body,
        grid=(grid,),
        in_specs=[
            pl.BlockSpec((NC, BN, 128), lambda i: (0, i, 0)),
            pl.BlockSpec((BN, 128), lambda i: (i, 0)),
            pl.BlockSpec((BN, 8), lambda i: (i, 0)),
            pl.BlockSpec((1, 128), lambda i: (0, 0)),
            pl.BlockSpec((128, 128), lambda i: (0, 0)),
        ],
        out_specs=pl.BlockSpec((BN, 128), lambda i: (i, 0)),
        out_shape=jax.ShapeDtypeStruct((n_pad, 128), jnp.float32),
    )


def _tc_final(n_pad, n_out):
    grid = n_pad // BN

    def body(p_ref, y2_ref, dinv8_ref, b2_ref, batch_ref, fw1_ref, fb1_ref,
             fw2_ref, fb2_ref, out_ref, sums, cnt):
        i = pl.program_id(0)

        @pl.when(i == 0)
        def _():
            sums[...] = jnp.zeros_like(sums)
            cnt[...] = jnp.zeros_like(cnt)

        dinv = dinv8_ref[:, 0:1]
        srow = (p_ref[0] + p_ref[1] + y2_ref[...]) * dinv
        h2 = jnp.maximum(srow + b2_ref[...], 0.0)

        row_ids = lax.broadcasted_iota(jnp.int32, (NG, BN), 0)
        bmat = jnp.broadcast_to(batch_ref[...], (NG, BN))
        mask = (row_ids == bmat).astype(jnp.float32)
        sums[...] += jnp.dot(mask, h2, preferred_element_type=jnp.float32)
        cnt[...] += jnp.sum(mask, axis=1, keepdims=True)

        @pl.when(i == grid - 1)
        def _():
            g = sums[...] / jnp.maximum(cnt[...], 1.0)
            z1 = jnp.maximum(
                jnp.dot(g, fw1_ref[...],
                        preferred_element_type=jnp.float32) + fb1_ref[...],
                0.0)
            z2 = jnp.dot(z1, fw2_ref[...],
                         preferred_element_type=jnp.float32) + fb2_ref[...]
            m = jnp.max(z2, axis=1, keepdims=True)
            e = z2 - m
            out_ref[...] = e - jnp.log(
                jnp.sum(jnp.exp(e), axis=1, keepdims=True))

    return pl.pallas_call(
        body,
        grid=(grid,),
        in_specs=[
            pl.BlockSpec((NC, BN, 128), lambda i: (0, i, 0)),
            pl.BlockSpec((BN, 128), lambda i: (i, 0)),
            pl.BlockSpec((BN, 8), lambda i: (i, 0)),
            pl.BlockSpec((1, 128), lambda i: (0, 0)),
            pl.BlockSpec((1, BN), lambda i: (0, i)),
            pl.BlockSpec((128, 128), lambda i: (0, 0)),
            pl.BlockSpec((1, 128), lambda i: (0, 0)),
            pl.BlockSpec((128, n_out), lambda i: (0, 0)),
            pl.BlockSpec((1, n_out), lambda i: (0, 0)),
        ],
        out_specs=pl.BlockSpec((NG, n_out), lambda i: (0, 0)),
        out_shape=jax.ShapeDtypeStruct((NG, n_out), jnp.float32),
        scratch_shapes=[
            pltpu.VMEM((NG, 128), jnp.float32),
            pltpu.VMEM((NG, 1), jnp.float32),
        ],
    )


def kernel(x, edge_index, batch, W1, b1, W2, b2, fW1, fb1, fW2, fb2):
    n, _ = x.shape
    e = edge_index.shape[1]
    n_out = fW2.shape[1]

    grp = NS * CH
    n_pad = -(-(n + 1) // (NS * CH)) * (NS * CH)
    n_pad = -(-n_pad // BN) * BN
    e_grp = NC * NS * CH * 2
    e_pad = -(-e // e_grp) * e_grp
    del grp

    pad_idx = n + (jnp.arange(e_pad - e, dtype=jnp.int32) % (n_pad - n))
    src_p = jnp.concatenate([edge_index[0], pad_idx])
    dst_p = jnp.concatenate([edge_index[1], pad_idx])
    x_pad = jnp.pad(x, ((0, n_pad - n), (0, 0)))
    batch2 = jnp.concatenate(
        [batch, jnp.full((n_pad - n,), NG, jnp.int32)]).reshape(1, n_pad)

    degp = _sc_degree(n_pad, e_pad)(dst_p)
    y1, dinv8 = _tc_first(n_pad)(x_pad, W1, degp)
    agg = _sc_aggregate(n_pad, e_pad)
    p1 = agg(y1, src_p, dst_p)
    y2 = _tc_mid(n_pad)(p1, y1, dinv8, b1.reshape(1, -1), W2)
    p2 = agg(y2, src_p, dst_p)
    out = _tc_final(n_pad, n_out)(
        p2, y2, dinv8, b2.reshape(1, -1), batch2, fW1, fb1.reshape(1, -1),
        fW2, fb2.reshape(1, -1))
    return out
